# baseline (speedup 1.0000x reference)
import functools

import jax
import jax.numpy as jnp
from jax import lax
from jax.experimental import pallas as pl
from jax.experimental.pallas import tpu as pltpu
from jax.experimental.pallas import tpu_sc as plsc

NC = 2
NS = 16
DEG = 32
D = 128
GROW = 128
NODES_PER_CHUNK = GROW // DEG
DCH = D // 16
NBUF = 4
NPASS = 2


def _agg_body(b_per_pass, nchunk, features, idx_all, out, idx_v, acc_v,
              *scratch):
    bufs = scratch[:NBUF]
    gsems = scratch[NBUF:2 * NBUF]
    osem = scratch[2 * NBUF]

    cid = lax.axis_index("c")
    tid = lax.axis_index("s")

    @pl.when(cid == 0)
    def _():
        pltpu.sync_copy(idx_all.at[tid], idx_v)

        def fire_gather(c, b):
            pltpu.async_copy(features.at[idx_v.at[c]], bufs[b], gsems[b])

        def wait_gather(c, b):
            pltpu.make_async_copy(features.at[idx_v.at[c]], bufs[b],
                                  gsems[b]).wait()

        def compute_chunk(c_local, b):
            buf = bufs[b]

            def node_body(n, carry):
                row0 = n * DEG
                for dc in range(DCH):
                    a = buf[row0, pl.ds(dc * 16, 16)]
                    for j in range(1, DEG):
                        a = a + buf[row0 + j, pl.ds(dc * 16, 16)]
                    acc_v[c_local * NODES_PER_CHUNK + n, pl.ds(dc * 16, 16)] = a
                return carry
            lax.fori_loop(0, NODES_PER_CHUNK, node_body, 0)

        for p in range(NPASS):
            base = p * nchunk
            for b in range(NBUF):
                fire_gather(base + b, b)

            @pl.when(p > 0)
            def _():
                pltpu.make_async_copy(
                    acc_v,
                    out.at[pl.ds((tid * NPASS + p - 1) * b_per_pass,
                                 b_per_pass)], osem).wait()

            def group_body(g, carry):
                for b in range(NBUF):
                    i = g * NBUF + b
                    wait_gather(base + i, b)
                    compute_chunk(i, b)

                    @pl.when(i + NBUF < nchunk)
                    def _():
                        fire_gather(base + i + NBUF, b)

                return carry

            lax.fori_loop(0, nchunk // NBUF, group_body, 0)

            pltpu.async_copy(
                acc_v,
                out.at[pl.ds((tid * NPASS + p) * b_per_pass, b_per_pass)],
                osem)

        pltpu.make_async_copy(
            acc_v,
            out.at[pl.ds((tid * NPASS + NPASS - 1) * b_per_pass, b_per_pass)],
            osem).wait()


def kernel(features, nodes, to_neighs):
    del nodes
    B = to_neighs.shape[0]
    tn = to_neighs.astype(jnp.int32)
    bp_unit = NS * NPASS * NODES_PER_CHUNK * NBUF * 2
    BP = ((B + bp_unit - 1) // bp_unit) * bp_unit
    b_per_pass = BP // (NS * NPASS)
    nchunk = b_per_pass * DEG // GROW
    if BP != B:
        tn = jnp.pad(tn, ((0, BP - B), (0, 0)))
    idx_all = tn.reshape(NS, NPASS * nchunk, GROW)

    mesh = plsc.VectorSubcoreMesh(core_axis_name="c", subcore_axis_name="s")
    run = pl.kernel(
        functools.partial(_agg_body, b_per_pass, nchunk),
        out_type=jax.ShapeDtypeStruct((BP, D), jnp.float32),
        mesh=mesh,
        scratch_types=(
            [pltpu.VMEM((NPASS * nchunk, GROW), jnp.int32)]
            + [pltpu.VMEM((b_per_pass, D), jnp.float32)]
            + [pltpu.VMEM((GROW, D), jnp.float32) for _ in range(NBUF)]
            + [pltpu.SemaphoreType.DMA for _ in range(NBUF + 1)]
        ),
    )
    out = run(features, idx_all)
    return out[:B]
